# baseline (speedup 1.0000x reference)
import dataclasses
import functools

import numpy as np
import jax
import jax.numpy as jnp
from jax import lax
from jax.experimental import pallas as pl
from jax.experimental.pallas import tpu as pltpu
from jax.experimental.pallas import tpu_sc as plsc

_T = 2016
_TS = 288
_TP = 2048
_F = 1024
_NF = _T // 2 + 1
_TOPK = 7
_VE = _TS * 9
_ROW_TILE = 128
_G = 8
_NEG = -3.0e38


@functools.lru_cache(maxsize=1)
def _dft_consts():
    t = np.arange(_TP, dtype=np.float64)[:, None]
    f = np.arange(_F, dtype=np.float64)[None, :]
    ang = 2.0 * np.pi * t * f / _T
    valid = (t < _T) & (f < _NF)
    c = np.where(valid, np.cos(ang), 0.0).astype(np.float32)
    s = np.where(valid, np.sin(ang), 0.0).astype(np.float32)
    w = np.full((1, _F), 2.0 / _T, dtype=np.float64)
    w[0, 0] = 1.0 / _T
    w[0, _NF - 1] = 1.0 / _T
    w[0, _NF:] = 0.0
    return c, s, w.astype(np.float32)


def _corr_body(nrows_real, wb_ref, q_ref, k_ref, c_ref, s_ref, w_ref, o_ref):
    hi = lax.Precision.HIGHEST
    f32 = jnp.float32

    def dot_tf(a, b):
        return lax.dot_general(a, b, (((1,), (0,)), ((), ())),
                               precision=hi, preferred_element_type=f32)

    def dot_ff(a, b):
        return lax.dot_general(a, b, (((1,), (1,)), ((), ())),
                               precision=hi, preferred_element_type=f32)

    cm = c_ref[...]
    sm = s_ref[...]
    q = q_ref[...]
    k = k_ref[...]
    qc = dot_tf(q, cm[0:_TS, :])
    qs = dot_tf(q, sm[0:_TS, :])
    kc = dot_tf(k, cm)
    ks = dot_tf(k, sm)
    w = w_ref[...]
    rr = (qc * kc + qs * ks) * w
    ri = (qc * ks - qs * kc) * w
    corr = dot_ff(rr, cm) - dot_ff(ri, sm)
    corr = corr * wb_ref[0] + wb_ref[1]
    col = lax.broadcasted_iota(jnp.int32, (_ROW_TILE, _TP), 1)
    corr = jnp.where(col >= _T, _NEG, corr)
    row = pl.program_id(0) * _ROW_TILE + lax.broadcasted_iota(
        jnp.int32, (_ROW_TILE, _TP), 0)
    corr = jnp.where(row >= nrows_real, -col.astype(f32), corr)
    o_ref[...] = corr


def _corr_pallas(wb, q2, k2, nrows_pad, nrows_real):
    c, s, w = _dft_consts()
    grid = (nrows_pad // _ROW_TILE,)
    return pl.pallas_call(
        functools.partial(_corr_body, nrows_real),
        grid=grid,
        in_specs=[
            pl.BlockSpec(memory_space=pltpu.SMEM),
            pl.BlockSpec((_ROW_TILE, _TS), lambda i: (i, 0)),
            pl.BlockSpec((_ROW_TILE, _TP), lambda i: (i, 0)),
            pl.BlockSpec((_TP, _F), lambda i: (0, 0)),
            pl.BlockSpec((_TP, _F), lambda i: (0, 0)),
            pl.BlockSpec((1, _F), lambda i: (0, 0)),
        ],
        out_specs=pl.BlockSpec((_ROW_TILE, _TP), lambda i: (i, 0)),
        out_shape=jax.ShapeDtypeStruct((nrows_pad, _TP), jnp.float32),
    )(wb, q2, k2, jnp.asarray(c), jnp.asarray(s), jnp.asarray(w))


_NSTEP = 12
_VN = _T * _NSTEP


def _topk_row(corr_v, delay_v, attn_v, cval, cidx, s):
    lanes = lax.iota(jnp.int32, 16)
    nchunk = _TP // 16
    neg16 = jnp.full((16,), _NEG, jnp.float32)

    def cm_body(c, m):
        return jnp.maximum(m, corr_v[s, pl.ds(c * 16, 16)])
    m = lax.fori_loop(0, nchunk, cm_body, neg16, unroll=8)

    ms, _ = plsc.sort_key_val(m, m, descending=True)
    tau = ms.at[jnp.zeros((16,), jnp.int32) + (_TOPK - 1)].get(
        mode="promise_in_bounds")

    def comp_body(c, cnt):
        v = corr_v[s, pl.ds(c * 16, 16)]
        msk = v >= tau
        mi = msk.astype(jnp.int32)
        pos = cnt + plsc.cumsum(mi) - 1
        plsc.store_scatter(cval, [pos], v, mask=msk)
        plsc.store_scatter(cidx, [pos], c * 16 + lanes, mask=msk)
        return cnt + plsc.all_reduce_population_count(msk)
    cntv = lax.fori_loop(0, nchunk, comp_body, jnp.zeros((16,), jnp.int32),
                         unroll=4)
    cnt = jnp.max(cntv)
    cval[pl.ds(cnt, 16)] = neg16

    nch = lax.div(cnt + 15, jnp.int32(16))
    big = jnp.int32(1 << 30)

    def sel_one(i, carry):
        dvec, vvec = carry

        def scan_body(c, st):
            mm, am = st
            v = cval[pl.ds(c * 16, 16)]
            upd = v > mm
            return jnp.where(upd, v, mm), jnp.where(upd, c, am)
        mm, am = lax.fori_loop(0, nch, scan_body,
                               (neg16, jnp.zeros((16,), jnp.int32)))
        bv = jnp.max(mm)
        pos = jnp.min(jnp.where(mm >= bv, am * 16 + lanes, big))
        cp = lax.div(pos, jnp.int32(16))
        lp = lax.rem(pos, jnp.int32(16))
        lpv = jnp.zeros((16,), jnp.int32) + lp
        civ = cidx[pl.ds(cp * 16, 16)]
        dsel = civ.at[lpv].get(mode="promise_in_bounds")
        vv = cval[pl.ds(cp * 16, 16)]
        cval[pl.ds(cp * 16, 16)] = jnp.where(lanes == lp, _NEG, vv)
        return (jnp.where(lanes == i, dsel, dvec),
                jnp.where(lanes == i, bv, vvec))

    dvec, vvec = lax.fori_loop(
        0, _TOPK, sel_one, (jnp.zeros((16,), jnp.int32), neg16))

    in7 = lanes < _TOPK
    vmax = jnp.max(jnp.where(in7, vvec, _NEG))
    e = jnp.where(in7, jnp.exp(jnp.where(in7, vvec - vmax, 0.0)), 0.0)
    att = e / jnp.sum(e)

    delay_v[s, :] = jnp.where(in7, dvec, 0)
    attn_v[s, :] = att


def _agg_row(v_blk, delay_v, attn_v, out_v, s):
    lanes = lax.iota(jnp.int32, 16)
    dvec = delay_v[s, :]
    att = attn_v[s, :]
    row0 = jnp.zeros((16,), jnp.int32)
    svec = row0 + s
    accs = [jnp.zeros((16,), jnp.float32) for _ in range(_TS // 16)]
    for i in range(_TOPK):
        iv = row0 + i
        dbc = dvec.at[iv].get(mode="promise_in_bounds")
        abc = att.at[iv].get(mode="promise_in_bounds")
        tt0 = dbc + _TS + lanes
        for u in range(_TS // 16):
            tt = tt0 + (u * 16)
            g = plsc.load_gather(
                v_blk, [row0, row0, jnp.minimum(tt, _T - 1), svec])
            accs[u] = accs[u] + abc * jnp.where(tt < _T, g, 0.0)
    for u in range(_TS // 16):
        out_v[s, pl.ds(u * 16, 16)] = accs[u]


def _sc_compiler_params(untiled):
    cp = pltpu.CompilerParams()
    flds = [("needs_layout_passes", False)]
    if untiled:
        flds.append(("use_tc_tiling_on_sc", False))
    for fld, val in flds:
        if fld in pltpu.CompilerParams.__dataclass_fields__:
            cp = dataclasses.replace(cp, **{fld: val})
    return cp


def _sc_topk(corr, nrows_pad):
    mesh = plsc.VectorSubcoreMesh(core_axis_name="c", subcore_axis_name="s")

    @functools.partial(
        pl.kernel,
        compiler_params=_sc_compiler_params(untiled=False),
        out_type=[
            jax.ShapeDtypeStruct((nrows_pad, 16), jnp.int32),
            jax.ShapeDtypeStruct((nrows_pad, 16), jnp.float32),
        ],
        mesh=mesh,
        scratch_types=[
            pltpu.VMEM((_TP + 16,), jnp.float32),
            pltpu.VMEM((_TP + 16,), jnp.int32),
        ],
    )
    def k(corr_hbm, delay_hbm, attn_hbm, cval, cidx):
        def body(corr_v, delay_v, attn_v):
            @pl.loop(0, 8)
            def _(s):
                _topk_row(corr_v, delay_v, attn_v, cval, cidx, s)

        pltpu.emit_pipeline(
            body,
            grid=(nrows_pad // 8,),
            in_specs=[pl.BlockSpec((8, _TP), lambda i: (i, 0))],
            out_specs=[
                pl.BlockSpec((8, 16), lambda i: (i, 0)),
                pl.BlockSpec((8, 16), lambda i: (i, 0)),
            ],
            core_axis_name=("c", "s"),
            dimension_semantics=(pltpu.PARALLEL,),
        )(corr_hbm, delay_hbm, attn_hbm)

    return k(corr)


def _sc_agg(values, delay16, attn16, nrows_pad):
    mesh = plsc.VectorSubcoreMesh(core_axis_name="c", subcore_axis_name="s")
    B, N = values.shape[0], values.shape[1]

    @functools.partial(
        pl.kernel,
        compiler_params=_sc_compiler_params(untiled=True),
        out_type=jax.ShapeDtypeStruct((nrows_pad, _TS), jnp.float32),
        mesh=mesh,
        scratch_types=[],
    )
    def k(v_hbm, delay_hbm, attn_hbm, out_hbm):
        def body(v_blk, delay_v, attn_v, out_v):
            @pl.loop(0, _NSTEP)
            def _(s):
                _agg_row(v_blk, delay_v, attn_v, out_v, s)

        pltpu.emit_pipeline(
            body,
            grid=(B * N,),
            in_specs=[
                pl.BlockSpec((1, 1, _T, _NSTEP),
                             lambda i: (i // N, i % N, 0, 0)),
                pl.BlockSpec((_NSTEP, 16), lambda i: (i, 0)),
                pl.BlockSpec((_NSTEP, 16), lambda i: (i, 0)),
            ],
            out_specs=[pl.BlockSpec((_NSTEP, _TS), lambda i: (i, 0))],
            core_axis_name=("c", "s"),
            dimension_semantics=(pltpu.PARALLEL,),
        )(v_hbm, delay_hbm, attn_hbm, out_hbm)

    return k(values, delay16, attn16)


def kernel(queries, keys, values, W, b):
    B, N, ts, S = queries.shape
    T = keys.shape[2]
    nbn = B * N
    nbn_pad = 832
    nrows = nbn * S
    nrows_pad = nbn_pad * S
    pad = nrows_pad - nrows

    q = queries.transpose(0, 1, 3, 2).reshape(nrows, ts)
    k = keys.transpose(0, 1, 3, 2).reshape(nrows, T)
    q2 = jnp.pad(q, ((0, pad), (0, 0)))
    k2 = jnp.pad(k, ((0, pad), (0, _TP - T)))
    wb = jnp.stack([W[0, 0], b[0]])

    corr = _corr_pallas(wb, q2, k2, nrows_pad, nrows)
    delay16, attn16 = _sc_topk(corr, nrows_pad)
    out48 = _sc_agg(values, delay16, attn16, nrows_pad)

    out = out48[:nrows].reshape(B, N, S, ts).transpose(0, 1, 3, 2)
    delay = delay16[:nrows, :_TOPK].reshape(B, N, S, _TOPK).transpose(0, 1, 3, 2)
    attn = attn16[:nrows, :_TOPK].reshape(B, N, S, _TOPK).transpose(0, 1, 3, 2)
    return out, delay, attn


# speedup vs baseline: 1.1508x; 1.1508x over previous
import dataclasses
import functools

import numpy as np
import jax
import jax.numpy as jnp
from jax import lax
from jax.experimental import pallas as pl
from jax.experimental.pallas import tpu as pltpu
from jax.experimental.pallas import tpu_sc as plsc

_T = 2016
_TS = 288
_TP = 2048
_F = 1024
_NF = _T // 2 + 1
_TOPK = 7
_VE = _TS * 9
_ROW_TILE = 192
_G = 8
_NEG = -3.0e38


@functools.lru_cache(maxsize=1)
def _dft_consts():
    t = np.arange(_TP, dtype=np.float64)[:, None]
    f = np.arange(_F, dtype=np.float64)[None, :]
    ang = 2.0 * np.pi * t * f / _T
    valid = (t < _T) & (f < _NF)
    c = np.where(valid, np.cos(ang), 0.0).astype(np.float32)
    s = np.where(valid, np.sin(ang), 0.0).astype(np.float32)
    w = np.full((1, _F), 2.0 / _T, dtype=np.float64)
    w[0, 0] = 1.0 / _T
    w[0, _NF - 1] = 1.0 / _T
    w[0, _NF:] = 0.0
    return c, s, w.astype(np.float32)


def _corr_body(nrows_real, wb_ref, q_ref, k_ref, c_ref, s_ref, w_ref, o_ref):
    hi = lax.Precision.HIGHEST
    f32 = jnp.float32

    def dot_tf(a, b):
        return lax.dot_general(a, b, (((1,), (0,)), ((), ())),
                               precision=hi, preferred_element_type=f32)

    def dot_ff(a, b):
        return lax.dot_general(a, b, (((1,), (1,)), ((), ())),
                               precision=hi, preferred_element_type=f32)

    cm = c_ref[...]
    sm = s_ref[...]
    q = q_ref[...]
    k = k_ref[...]
    qc = dot_tf(q, cm[0:_TS, :])
    qs = dot_tf(q, sm[0:_TS, :])
    kc = dot_tf(k, cm)
    ks = dot_tf(k, sm)
    w = w_ref[...]
    rr = (qc * kc + qs * ks) * w
    ri = (qc * ks - qs * kc) * w
    corr = dot_ff(rr, cm) - dot_ff(ri, sm)
    corr = corr * wb_ref[0] + wb_ref[1]
    col = lax.broadcasted_iota(jnp.int32, (_ROW_TILE, _TP), 1)
    corr = jnp.where(col >= _T, _NEG, corr)
    row = pl.program_id(0) * _ROW_TILE + lax.broadcasted_iota(
        jnp.int32, (_ROW_TILE, _TP), 0)
    corr = jnp.where(row >= nrows_real, -col.astype(f32), corr)
    o_ref[...] = corr


def _corr_pallas(wb, q2, k2, nrows_pad, nrows_real):
    c, s, w = _dft_consts()
    grid = (nrows_pad // _ROW_TILE,)
    return pl.pallas_call(
        functools.partial(_corr_body, nrows_real),
        grid=grid,
        in_specs=[
            pl.BlockSpec(memory_space=pltpu.SMEM),
            pl.BlockSpec((_ROW_TILE, _TS), lambda i: (i, 0)),
            pl.BlockSpec((_ROW_TILE, _TP), lambda i: (i, 0)),
            pl.BlockSpec((_TP, _F), lambda i: (0, 0)),
            pl.BlockSpec((_TP, _F), lambda i: (0, 0)),
            pl.BlockSpec((1, _F), lambda i: (0, 0)),
        ],
        out_specs=pl.BlockSpec((_ROW_TILE, _TP), lambda i: (i, 0)),
        out_shape=jax.ShapeDtypeStruct((nrows_pad, _TP), jnp.float32),
    )(wb, q2, k2, jnp.asarray(c), jnp.asarray(s), jnp.asarray(w))


_NSTEP = 12
_VN = _T * _NSTEP


def _topk_row(corr_v, delay_v, attn_v, cval, cidx, s):
    lanes = lax.iota(jnp.int32, 16)
    nchunk = _TP // 16
    neg16 = jnp.full((16,), _NEG, jnp.float32)

    def cm_body(c, m):
        return jnp.maximum(m, corr_v[s, pl.ds(c * 16, 16)])
    m = lax.fori_loop(0, nchunk, cm_body, neg16, unroll=8)

    ms, _ = plsc.sort_key_val(m, m, descending=True)
    tau = ms.at[jnp.zeros((16,), jnp.int32) + (_TOPK - 1)].get(
        mode="promise_in_bounds")

    def comp_body(c, cnt):
        v = corr_v[s, pl.ds(c * 16, 16)]
        msk = v >= tau
        mi = msk.astype(jnp.int32)
        pos = cnt + plsc.cumsum(mi) - 1
        plsc.store_scatter(cval, [pos], v, mask=msk)
        plsc.store_scatter(cidx, [pos], c * 16 + lanes, mask=msk)
        return cnt + plsc.all_reduce_population_count(msk)
    cntv = lax.fori_loop(0, nchunk, comp_body, jnp.zeros((16,), jnp.int32),
                         unroll=4)
    cnt = jnp.max(cntv)
    cval[pl.ds(cnt, 16)] = neg16

    nch = lax.div(cnt + 15, jnp.int32(16))
    big = jnp.int32(1 << 30)

    def sel_one(i, carry):
        dvec, vvec = carry

        def scan_body(c, st):
            mm, am = st
            v = cval[pl.ds(c * 16, 16)]
            upd = v > mm
            return jnp.where(upd, v, mm), jnp.where(upd, c, am)
        mm, am = lax.fori_loop(0, nch, scan_body,
                               (neg16, jnp.zeros((16,), jnp.int32)))
        bv = jnp.max(mm)
        pos = jnp.min(jnp.where(mm >= bv, am * 16 + lanes, big))
        cp = lax.div(pos, jnp.int32(16))
        lp = lax.rem(pos, jnp.int32(16))
        lpv = jnp.zeros((16,), jnp.int32) + lp
        civ = cidx[pl.ds(cp * 16, 16)]
        dsel = civ.at[lpv].get(mode="promise_in_bounds")
        vv = cval[pl.ds(cp * 16, 16)]
        cval[pl.ds(cp * 16, 16)] = jnp.where(lanes == lp, _NEG, vv)
        return (jnp.where(lanes == i, dsel, dvec),
                jnp.where(lanes == i, bv, vvec))

    dvec, vvec = lax.fori_loop(
        0, _TOPK, sel_one, (jnp.zeros((16,), jnp.int32), neg16))

    in7 = lanes < _TOPK
    vmax = jnp.max(jnp.where(in7, vvec, _NEG))
    e = jnp.where(in7, jnp.exp(jnp.where(in7, vvec - vmax, 0.0)), 0.0)
    att = e / jnp.sum(e)

    delay_v[s, :] = jnp.where(in7, dvec, 0)
    attn_v[s, :] = att


def _agg_row(v_blk, delay_v, attn_v, out_v, s):
    lanes = lax.iota(jnp.int32, 16)
    dvec = delay_v[s, :]
    att = attn_v[s, :]
    row0 = jnp.zeros((16,), jnp.int32)
    svec = row0 + s
    accs = [jnp.zeros((16,), jnp.float32) for _ in range(_TS // 16)]
    for i in range(_TOPK):
        iv = row0 + i
        dbc = dvec.at[iv].get(mode="promise_in_bounds")
        abc = att.at[iv].get(mode="promise_in_bounds")
        tt0 = dbc + _TS + lanes
        for u in range(_TS // 16):
            tt = tt0 + (u * 16)
            g = plsc.load_gather(
                v_blk, [row0, row0, jnp.minimum(tt, _T - 1), svec])
            accs[u] = accs[u] + abc * jnp.where(tt < _T, g, 0.0)
    for u in range(_TS // 16):
        out_v[s, pl.ds(u * 16, 16)] = accs[u]


def _sc_compiler_params(untiled):
    cp = pltpu.CompilerParams()
    flds = [("needs_layout_passes", False)]
    if untiled:
        flds.append(("use_tc_tiling_on_sc", False))
    for fld, val in flds:
        if fld in pltpu.CompilerParams.__dataclass_fields__:
            cp = dataclasses.replace(cp, **{fld: val})
    return cp


def _sc_topk(corr, nrows_pad):
    mesh = plsc.VectorSubcoreMesh(core_axis_name="c", subcore_axis_name="s")

    @functools.partial(
        pl.kernel,
        compiler_params=_sc_compiler_params(untiled=False),
        out_type=[
            jax.ShapeDtypeStruct((nrows_pad, 16), jnp.int32),
            jax.ShapeDtypeStruct((nrows_pad, 16), jnp.float32),
        ],
        mesh=mesh,
        scratch_types=[
            pltpu.VMEM((_TP + 16,), jnp.float32),
            pltpu.VMEM((_TP + 16,), jnp.int32),
        ],
    )
    def k(corr_hbm, delay_hbm, attn_hbm, cval, cidx):
        def body(corr_v, delay_v, attn_v):
            @pl.loop(0, 8)
            def _(s):
                _topk_row(corr_v, delay_v, attn_v, cval, cidx, s)

        pltpu.emit_pipeline(
            body,
            grid=(nrows_pad // 8,),
            in_specs=[pl.BlockSpec((8, _TP), lambda i: (i, 0))],
            out_specs=[
                pl.BlockSpec((8, 16), lambda i: (i, 0)),
                pl.BlockSpec((8, 16), lambda i: (i, 0)),
            ],
            core_axis_name=("c", "s"),
            dimension_semantics=(pltpu.PARALLEL,),
        )(corr_hbm, delay_hbm, attn_hbm)

    return k(corr)


def _sc_agg(values, delay16, attn16, nrows_pad):
    mesh = plsc.VectorSubcoreMesh(core_axis_name="c", subcore_axis_name="s")
    B, N = values.shape[0], values.shape[1]

    @functools.partial(
        pl.kernel,
        compiler_params=_sc_compiler_params(untiled=True),
        out_type=jax.ShapeDtypeStruct((nrows_pad, _TS), jnp.float32),
        mesh=mesh,
        scratch_types=[],
    )
    def k(v_hbm, delay_hbm, attn_hbm, out_hbm):
        def body(v_blk, delay_v, attn_v, out_v):
            @pl.loop(0, _NSTEP)
            def _(s):
                _agg_row(v_blk, delay_v, attn_v, out_v, s)

        pltpu.emit_pipeline(
            body,
            grid=(B * N,),
            in_specs=[
                pl.BlockSpec((1, 1, _T, _NSTEP),
                             lambda i: (i // N, i % N, 0, 0)),
                pl.BlockSpec((_NSTEP, 16), lambda i: (i, 0)),
                pl.BlockSpec((_NSTEP, 16), lambda i: (i, 0)),
            ],
            out_specs=[pl.BlockSpec((_NSTEP, _TS), lambda i: (i, 0))],
            core_axis_name=("c", "s"),
            dimension_semantics=(pltpu.PARALLEL,),
        )(v_hbm, delay_hbm, attn_hbm, out_hbm)

    return k(values, delay16, attn16)


def kernel(queries, keys, values, W, b):
    B, N, ts, S = queries.shape
    T = keys.shape[2]
    nbn = B * N
    nbn_pad = 832
    nrows = nbn * S
    nrows_pad = nbn_pad * S
    pad = nrows_pad - nrows

    q = queries.transpose(0, 1, 3, 2).reshape(nrows, ts)
    k = keys.transpose(0, 1, 3, 2).reshape(nrows, T)
    q2 = jnp.pad(q, ((0, pad), (0, 0)))
    k2 = jnp.pad(k, ((0, pad), (0, _TP - T)))
    wb = jnp.stack([W[0, 0], b[0]])

    corr = _corr_pallas(wb, q2, k2, nrows_pad, nrows)
    delay16, attn16 = _sc_topk(corr, nrows_pad)
    out48 = _sc_agg(values, delay16, attn16, nrows_pad)

    out = out48[:nrows].reshape(B, N, S, ts).transpose(0, 1, 3, 2)
    delay = delay16[:nrows, :_TOPK].reshape(B, N, S, _TOPK).transpose(0, 1, 3, 2)
    attn = attn16[:nrows, :_TOPK].reshape(B, N, S, _TOPK).transpose(0, 1, 3, 2)
    return out, delay, attn
